# baseline (speedup 1.0000x reference)
import jax
import jax.numpy as jnp
from jax import lax
from jax.experimental import pallas as pl
from jax.experimental.pallas import tpu as pltpu
from jax.experimental.pallas import tpu_sc as plsc

_N = 16384
_TABLE = 10
_NS = 16
_L = 16
_CHUNK = _N // _NS


def _lookup_body(y_hbm, table_hbm, out_hbm, buf_v, table_v,
                 sem_t, sem_y0, sem_y1, sem_o0):
    wid = lax.axis_index("s")
    base = wid * _CHUNK
    half = _CHUNK // 2
    cp_t = pltpu.make_async_copy(table_hbm, table_v, sem_t)
    cp_y0 = pltpu.make_async_copy(
        y_hbm.at[pl.ds(base, half)], buf_v.at[pl.ds(0, half)], sem_y0)
    cp_y1 = pltpu.make_async_copy(
        y_hbm.at[pl.ds(base + half, half)], buf_v.at[pl.ds(half, half)], sem_y1)
    cp_y0.start()
    cp_y1.start()
    cp_t.start()

    def gather_half(off):
        def step(i):
            idx = buf_v[pl.ds(off + i, _L)]
            buf_v[pl.ds(off + i, _L)] = plsc.load_gather(table_v, [idx])
        plsc.parallel_loop(0, half, _L, unroll=8)(step)

    cp_y0.wait()
    cp_t.wait()
    gather_half(0)
    cp_o0 = pltpu.make_async_copy(
        buf_v.at[pl.ds(0, half)], out_hbm.at[pl.ds(base, half)], sem_o0)
    cp_o0.start()
    cp_y1.wait()
    gather_half(half)
    pltpu.sync_copy(
        buf_v.at[pl.ds(half, half)], out_hbm.at[pl.ds(base + half, half)])
    cp_o0.wait()


def kernel(y, table):
    run = pl.kernel(
        _lookup_body,
        out_type=jax.ShapeDtypeStruct((_N,), jnp.int32),
        mesh=plsc.VectorSubcoreMesh(
            core_axis_name="c", subcore_axis_name="s", num_cores=1,
            num_subcores=_NS,
        ),
        compiler_params=pltpu.CompilerParams(needs_layout_passes=False),
        scratch_types=[
            pltpu.VMEM((_CHUNK,), jnp.int32),
            pltpu.VMEM((_TABLE,), jnp.int32),
            pltpu.SemaphoreType.DMA,
            pltpu.SemaphoreType.DMA,
            pltpu.SemaphoreType.DMA,
            pltpu.SemaphoreType.DMA,
        ],
    )
    return run(y.astype(jnp.int32), table.astype(jnp.int32))


# speedup vs baseline: 1.0149x; 1.0149x over previous
import jax
import jax.numpy as jnp
from jax import lax
from jax.experimental import pallas as pl
from jax.experimental.pallas import tpu as pltpu
from jax.experimental.pallas import tpu_sc as plsc

_N = 16384
_TABLE = 10
_NS = 16
_L = 16
_CHUNK = _N // _NS


def _lookup_body(y_hbm, table_hbm, out_hbm, buf_v, table_v,
                 sem_t, sem_y0, sem_y1, sem_o0):
    wid = lax.axis_index("s")
    base = wid * _CHUNK
    half = _CHUNK // 2
    cp_t = pltpu.make_async_copy(table_hbm, table_v, sem_t)
    cp_y0 = pltpu.make_async_copy(
        y_hbm.at[pl.ds(base, half)], buf_v.at[pl.ds(0, half)], sem_y0)
    cp_y1 = pltpu.make_async_copy(
        y_hbm.at[pl.ds(base + half, half)], buf_v.at[pl.ds(half, half)], sem_y1)
    cp_y0.start()
    cp_y1.start()
    cp_t.start()

    def gather_half(off):
        def step(i):
            idx = buf_v[pl.ds(off + i, _L)]
            buf_v[pl.ds(off + i, _L)] = plsc.load_gather(table_v, [idx])
        plsc.parallel_loop(0, half, _L, unroll=4)(step)

    cp_y0.wait()
    cp_t.wait()
    gather_half(0)
    cp_o0 = pltpu.make_async_copy(
        buf_v.at[pl.ds(0, half)], out_hbm.at[pl.ds(base, half)], sem_o0)
    cp_o0.start()
    cp_y1.wait()
    gather_half(half)
    pltpu.sync_copy(
        buf_v.at[pl.ds(half, half)], out_hbm.at[pl.ds(base + half, half)])
    cp_o0.wait()


def kernel(y, table):
    run = pl.kernel(
        _lookup_body,
        out_type=jax.ShapeDtypeStruct((_N,), jnp.int32),
        mesh=plsc.VectorSubcoreMesh(
            core_axis_name="c", subcore_axis_name="s", num_cores=1,
            num_subcores=_NS,
        ),
        compiler_params=pltpu.CompilerParams(needs_layout_passes=False),
        scratch_types=[
            pltpu.VMEM((_CHUNK,), jnp.int32),
            pltpu.VMEM((_TABLE,), jnp.int32),
            pltpu.SemaphoreType.DMA,
            pltpu.SemaphoreType.DMA,
            pltpu.SemaphoreType.DMA,
            pltpu.SemaphoreType.DMA,
        ],
    )
    return run(y.astype(jnp.int32), table.astype(jnp.int32))
